# baseline (speedup 1.0000x reference)
import functools

import jax
import jax.numpy as jnp
from jax import lax
from jax.experimental import pallas as pl
from jax.experimental.pallas import tpu as pltpu
from jax.experimental.pallas import tpu_sc as plsc

ALPHA = 0.5
N_CLASS = 100000
N_FEAT = 64
N_BATCH = 16384

L = 16
W = 8
NC = 2
NS = 16
G = 128
SC_ROWS = N_BATCH // NS
MY_ROWS = N_BATCH // (NC * NS)
NGRP = SC_ROWS // G
MYGRP = MY_ROWS // G
NROUND = N_FEAT // W
NPAIR = N_FEAT // L


CS = 16
NB = MY_ROWS // CS


def _body(feat_hbm, lab_hbm, cent_hbm, zo_hbm, out_hbm,
          lab2d, lab2, c_loc, cstage, f8, S, racc, beta, gamma,
          zo_b, tab, sem, semt):
    cid = lax.axis_index("c")
    sid = lax.axis_index("s")
    tile_base = sid * SC_ROWS
    my_base = tile_base + cid * MY_ROWS
    mygrp0 = cid * MYGRP

    lane = lax.iota(jnp.int32, L)
    czero = jnp.zeros((L,), jnp.int32)

    lds = [pltpu.async_copy(lab_hbm.at[pl.ds(tile_base + g * G, G)],
                            lab2d.at[g], sem) for g in range(NGRP)]
    z0 = pltpu.async_copy(zo_hbm.at[0], zo_b, sem)
    for d in lds:
        d.wait()
    z0.wait()

    zds = [pltpu.async_copy(zo_b, tab.at[lab2d.at[g]], semt)
           for g in range(NGRP)]

    def _mklab2(i, _):
        g = i // W
        o = (i % W) * L
        v = lab2d[mygrp0 + g, pl.ds(o, L)]
        lab2[g // MYGRP, pl.ds((g % MYGRP) * G + o, L)] = v // 2
        return 0
    lax.fori_loop(0, MY_ROWS // L, _mklab2, 0)

    pltpu.async_copy(cent_hbm.at[lab2.at[0, pl.ds(0, CS)]],
                     cstage.at[0], sem)

    def _cent_pipe(b, _):
        @pl.when(b + 1 < NB)
        def _fire_next():
            pltpu.async_copy(
                cent_hbm.at[lab2.at[0, pl.ds((b + 1) * CS, CS)]],
                cstage.at[(b + 1) % 2], sem)
        pltpu.make_async_copy(cent_hbm.at[pl.ds(0, CS)],
                              cstage.at[b % 2], sem).wait()
        bsel = czero + b % 2
        lab16 = lab2d[mygrp0 + b // W, pl.ds((b % W) * L, L)]
        coff16 = (lab16 % 2) * N_FEAT
        for j in range(L):
            srow = czero + j
            for u in range(N_FEAT // L):
                v = plsc.load_gather(
                    cstage, [bsel, srow, coff16[j] + u * L + lane])
                c_loc[b * L + j, pl.ds(u * L, L)] = v
        return 0
    lax.fori_loop(0, NB, _cent_pipe, 0)

    for d in zds:
        d.wait()
    z1 = pltpu.async_copy(zo_hbm.at[1], zo_b, sem)
    z1.wait()
    plsc.subcore_barrier()

    ads = [pltpu.async_copy(zo_b, tab.at[lab2d.at[g]], semt, add=True)
           for g in range(NGRP)]
    for d in ads:
        d.wait()
    plsc.subcore_barrier()
    gds = [pltpu.async_copy(tab.at[lab2d.at[mygrp0 + g]],
                            S.at[2, pl.ds(g * G, G), :], semt)
           for g in range(MYGRP)]
    fd = pltpu.async_copy(
        feat_hbm.at[pl.ds(tile_base, SC_ROWS), pl.ds(0, W)], f8.at[0], sem)
    for d in gds:
        d.wait()

    ctwo = jnp.full((L,), 2, jnp.int32)

    def _coef(b, _):
        n16 = plsc.load_gather(S, [ctwo, b * L + lane, czero])
        d = 1.0 / (1.0 + n16)
        beta[pl.ds(b * L, L)] = 1.0 - ALPHA * n16 * d
        gamma[pl.ds(b * L, L)] = ALPHA * d
        return 0
    lax.fori_loop(0, MY_ROWS // L, _coef, 0)
    plsc.subcore_barrier()

    hsel = lane // W
    wsel = lane % W

    def _fire_feat(h):
        return pltpu.async_copy(
            feat_hbm.at[pl.ds(tile_base, SC_ROWS), pl.ds(h * W, W)],
            f8.at[h % 2], sem)

    for h in range(NROUND):
        cur = h % 3
        if h % 2 == 0 and h + 1 < NROUND:
            fd_next = _fire_feat(h + 1)
        fd.wait()
        ads = [pltpu.async_copy(f8.at[h % 2, pl.ds(g * G, G), :],
                                tab.at[lab2d.at[g]], semt, add=True)
               for g in range(NGRP)]
        for d in ads:
            d.wait()
        plsc.subcore_barrier()
        gds = [pltpu.async_copy(tab.at[lab2d.at[mygrp0 + g]],
                                S.at[cur, pl.ds(g * G, G), :], semt)
               for g in range(MYGRP)]
        for d in gds:
            d.wait()
        plsc.subcore_barrier()

        if h % 2 == 1:
            t = h // 2
            b1 = (2 * t) % 3
            b2 = (2 * t + 1) % 3
            bp = (2 * t - 1) % 3
            ssel = czero + b1 + hsel * (b2 - b1)
            psel = czero + bp + hsel * (b1 - bp)

            def _comp(b, _):
                b16 = beta[pl.ds(b * L, L)]
                g16 = gamma[pl.ds(b * L, L)]
                for j in range(L):
                    s = b * L + j
                    srow = czero + s
                    f = plsc.load_gather(
                        f8, [hsel, srow + cid * MY_ROWS, wsel])
                    sc = plsc.load_gather(S, [ssel, srow, wsel])
                    sp = plsc.load_gather(S, [psel, srow, wsel])
                    c = c_loc[s, pl.ds(t * L, L)]
                    d = f - b16[j] * c - g16[j] * (sc - sp)
                    if t == 0:
                        racc[s, :] = d * d
                    else:
                        racc[s, :] = racc[s, :] + d * d
                return 0
            lax.fori_loop(0, MY_ROWS // L, _comp, 0)
            if h + 1 < NROUND:
                fd_next = _fire_feat(h + 1)
        if h + 1 < NROUND:
            fd = fd_next

    def _fin(b, _):
        rows = b * L + lane
        acc = plsc.load_gather(racc, [rows, czero])
        for j in range(1, L):
            acc = acc + plsc.load_gather(racc,
                                         [rows, jnp.full((L,), j, jnp.int32)])
        beta[pl.ds(b * L, L)] = 0.5 * acc
        return 0
    lax.fori_loop(0, MY_ROWS // L, _fin, 0)
    pltpu.sync_copy(beta, out_hbm.at[pl.ds(my_base, MY_ROWS)])


@functools.cache
def _build():
    return functools.partial(
        pl.kernel,
        out_type=jax.ShapeDtypeStruct((N_BATCH,), jnp.float32),
        compiler_params=pltpu.CompilerParams(use_tc_tiling_on_sc=False,
                                             needs_layout_passes=False),
        mesh=plsc.VectorSubcoreMesh(core_axis_name="c", subcore_axis_name="s",
                                    num_cores=NC, num_subcores=NS),
        scratch_types=[
            pltpu.VMEM((NGRP, G), jnp.int32),
            pltpu.VMEM((1, MY_ROWS), jnp.int32),
            pltpu.VMEM((MY_ROWS, N_FEAT), jnp.float32),
            pltpu.VMEM((2, CS, 2 * N_FEAT), jnp.float32),
            pltpu.VMEM((2, SC_ROWS, W), jnp.float32),
            pltpu.VMEM((3, MY_ROWS, W), jnp.float32),
            pltpu.VMEM((MY_ROWS, L), jnp.float32),
            pltpu.VMEM((MY_ROWS,), jnp.float32),
            pltpu.VMEM((MY_ROWS,), jnp.float32),
            pltpu.VMEM((G, W), jnp.float32),
            pltpu.VMEM_SHARED((N_CLASS, W), jnp.float32),
            pltpu.SemaphoreType.DMA,
            pltpu.SemaphoreType.DMA,
        ],
    )(_body)


def kernel(features, labels, centers):
    labels = jnp.reshape(labels, (-1,)).astype(jnp.int32)
    zo = jnp.stack([jnp.zeros((G, W), jnp.float32),
                    jnp.ones((G, W), jnp.float32)])
    cent2 = centers.reshape(N_CLASS // 2, 2 * N_FEAT)
    return jnp.reshape(_build()(features, labels, cent2, zo), (N_BATCH, 1))


# speedup vs baseline: 1.0802x; 1.0802x over previous
import functools

import jax
import jax.numpy as jnp
from jax import lax
from jax.experimental import pallas as pl
from jax.experimental.pallas import tpu as pltpu
from jax.experimental.pallas import tpu_sc as plsc

ALPHA = 0.5
N_CLASS = 100000
N_FEAT = 64
N_BATCH = 16384

L = 16
W = 8
NC = 2
NS = 16
G = 128
SC_ROWS = N_BATCH // NS
MY_ROWS = N_BATCH // (NC * NS)
NGRP = SC_ROWS // G
MYGRP = MY_ROWS // G
NROUND = N_FEAT // W
NPAIR = N_FEAT // L


def _body(feat_hbm, lab_hbm, cent_hbm, zo_hbm, out_hbm,
          lab2d, labp, c_loc, f8, S, racc, beta, gamma,
          zo_b, tab, sem, semt):
    cid = lax.axis_index("c")
    sid = lax.axis_index("s")
    tile_base = sid * SC_ROWS
    my_base = tile_base + cid * MY_ROWS
    mygrp0 = cid * MYGRP

    lane = lax.iota(jnp.int32, L)
    czero = jnp.zeros((L,), jnp.int32)

    lds = [pltpu.async_copy(lab_hbm.at[pl.ds(tile_base + g * G, G)],
                            lab2d.at[g], sem) for g in range(NGRP)]
    z0 = pltpu.async_copy(zo_hbm.at[0], zo_b, sem)
    for d in lds:
        d.wait()
    z0.wait()

    def _mklabp(i, _):
        g = i // W
        k = i % W
        row = czero + 2 * g + k // MYGRP
        col = (k % MYGRP) * 2 * L + 2 * lane
        labp[0, g, pl.ds(k * L, L)] = plsc.load_gather(lab2d, [row, col])
        labp[1, g, pl.ds(k * L, L)] = plsc.load_gather(lab2d, [row, col + 1])
        return 0
    lax.fori_loop(0, MYGRP * W, _mklabp, 0)

    cds = [pltpu.async_copy(cent_hbm.at[lab2d.at[mygrp0 + g]],
                            c_loc.at[pl.ds(g * G, G)], sem)
           for g in range(MYGRP)]

    zds = [pltpu.async_copy(zo_b, tab.at[lab2d.at[g]], semt)
           for g in range(NGRP)]
    for d in zds:
        d.wait()
    for d in cds:
        d.wait()
    z1 = pltpu.async_copy(zo_hbm.at[1], zo_b, sem)
    z1.wait()
    plsc.subcore_barrier()

    ads = [pltpu.async_copy(zo_b, tab.at[lab2d.at[g]], semt, add=True)
           for g in range(NGRP)]
    for d in ads:
        d.wait()
    plsc.subcore_barrier()
    gds = [pltpu.async_copy(tab.at[lab2d.at[mygrp0 + g]],
                            S.at[2, pl.ds(g * G, G), :], semt)
           for g in range(MYGRP)]

    def _fire_feat(h):
        return [pltpu.async_copy(
            feat_hbm.at[pl.ds(tile_base // 2, SC_ROWS // 2),
                        pl.ds(p * N_FEAT + h * W, W)],
            f8.at[h % 2, p], sem) for p in range(2)]

    fd = _fire_feat(0)
    for d in gds:
        d.wait()

    ctwo = jnp.full((L,), 2, jnp.int32)

    def _coef(b, _):
        n16 = plsc.load_gather(S, [ctwo, b * L + lane, czero])
        d = 1.0 / (1.0 + n16)
        beta[pl.ds(b * L, L)] = 1.0 - ALPHA * n16 * d
        gamma[pl.ds(b * L, L)] = ALPHA * d
        return 0
    lax.fori_loop(0, MY_ROWS // L, _coef, 0)
    plsc.subcore_barrier()

    hsel = lane // W
    wsel = lane % W

    for h in range(NROUND):
        cur = h % 3
        if h % 2 == 0 and h + 1 < NROUND:
            fd_next = _fire_feat(h + 1)
        for d in fd:
            d.wait()
        ads = [pltpu.async_copy(f8.at[h % 2, p, pl.ds(g * G, G), :],
                                tab.at[labp.at[p, g]], semt, add=True)
               for p in range(2) for g in range(MYGRP)]
        for d in ads:
            d.wait()
        plsc.subcore_barrier()
        gds = [pltpu.async_copy(tab.at[lab2d.at[mygrp0 + g]],
                                S.at[cur, pl.ds(g * G, G), :], semt)
               for g in range(MYGRP)]
        for d in gds:
            d.wait()
        plsc.subcore_barrier()

        if h % 2 == 1:
            t = h // 2
            b1 = (2 * t) % 3
            b2 = (2 * t + 1) % 3
            bp = (2 * t - 1) % 3
            ssel = czero + b1 + hsel * (b2 - b1)
            psel = czero + bp + hsel * (b1 - bp)

            def _comp(b, _):
                b16 = beta[pl.ds(b * L, L)]
                g16 = gamma[pl.ds(b * L, L)]
                for j in range(L):
                    s = b * L + j
                    srow = czero + s
                    f = plsc.load_gather(
                        f8, [hsel, czero + j % 2,
                             czero + cid * (MY_ROWS // 2) + b * W + j // 2,
                             wsel])
                    sc = plsc.load_gather(S, [ssel, srow, wsel])
                    sp = plsc.load_gather(S, [psel, srow, wsel])
                    c = c_loc[s, pl.ds(t * L, L)]
                    d = f - b16[j] * c - g16[j] * (sc - sp)
                    if t == 0:
                        racc[s, :] = d * d
                    else:
                        racc[s, :] = racc[s, :] + d * d
                return 0
            lax.fori_loop(0, MY_ROWS // L, _comp, 0)
            if h + 1 < NROUND:
                fd_next = _fire_feat(h + 1)
        if h + 1 < NROUND:
            fd = fd_next

    def _fin(b, _):
        rows = b * L + lane
        acc = plsc.load_gather(racc, [rows, czero])
        for j in range(1, L):
            acc = acc + plsc.load_gather(racc,
                                         [rows, jnp.full((L,), j, jnp.int32)])
        beta[pl.ds(b * L, L)] = 0.5 * acc
        return 0
    lax.fori_loop(0, MY_ROWS // L, _fin, 0)
    pltpu.sync_copy(beta, out_hbm.at[pl.ds(my_base, MY_ROWS)])


@functools.cache
def _build():
    return functools.partial(
        pl.kernel,
        out_type=jax.ShapeDtypeStruct((N_BATCH,), jnp.float32),
        compiler_params=pltpu.CompilerParams(use_tc_tiling_on_sc=False,
                                             needs_layout_passes=False),
        mesh=plsc.VectorSubcoreMesh(core_axis_name="c", subcore_axis_name="s",
                                    num_cores=NC, num_subcores=NS),
        scratch_types=[
            pltpu.VMEM((NGRP, G), jnp.int32),
            pltpu.VMEM((2, MYGRP, G), jnp.int32),
            pltpu.VMEM((MY_ROWS, N_FEAT), jnp.float32),
            pltpu.VMEM((2, 2, SC_ROWS // 2, W), jnp.float32),
            pltpu.VMEM((3, MY_ROWS, W), jnp.float32),
            pltpu.VMEM((MY_ROWS, L), jnp.float32),
            pltpu.VMEM((MY_ROWS,), jnp.float32),
            pltpu.VMEM((MY_ROWS,), jnp.float32),
            pltpu.VMEM((G, W), jnp.float32),
            pltpu.VMEM_SHARED((N_CLASS, W), jnp.float32),
            pltpu.SemaphoreType.DMA,
            pltpu.SemaphoreType.DMA,
        ],
    )(_body)


def kernel(features, labels, centers):
    labels = jnp.reshape(labels, (-1,)).astype(jnp.int32)
    zo = jnp.stack([jnp.zeros((G, W), jnp.float32),
                    jnp.ones((G, W), jnp.float32)])
    f2 = features.reshape(N_BATCH // 2, 2 * N_FEAT)
    return jnp.reshape(_build()(f2, labels, centers, zo), (N_BATCH, 1))


# speedup vs baseline: 1.2057x; 1.1162x over previous
import functools

import jax
import jax.numpy as jnp
from jax import lax
from jax.experimental import pallas as pl
from jax.experimental.pallas import tpu as pltpu
from jax.experimental.pallas import tpu_sc as plsc

ALPHA = 0.5
N_CLASS = 100000
N_FEAT = 64
N_BATCH = 16384

L = 16
W = 8
NC = 2
NS = 16
G = 128
SC_ROWS = N_BATCH // NS
MY_ROWS = N_BATCH // (NC * NS)
NGRP = SC_ROWS // G
MYGRP = MY_ROWS // G
NROUND = N_FEAT // W
NPAIR = N_FEAT // L


def _body_a(feat_hbm, lab_hbm, zo_hbm, u_hbm, beta_hbm,
            lab2d, labp, f8, S, uacc, beta, gamma, zo_b, tab, sem, semt,
            semu):
    cid = lax.axis_index("c")
    sid = lax.axis_index("s")
    tile_base = sid * SC_ROWS
    my_base = tile_base + cid * MY_ROWS
    mygrp0 = cid * MYGRP

    lane = lax.iota(jnp.int32, L)
    czero = jnp.zeros((L,), jnp.int32)

    lds = [pltpu.async_copy(lab_hbm.at[pl.ds(tile_base + g * G, G)],
                            lab2d.at[g], sem) for g in range(NGRP)]
    z0 = pltpu.async_copy(zo_hbm.at[0], zo_b, sem)
    for d in lds:
        d.wait()
    z0.wait()

    def _mklabp(i, _):
        g = i // W
        k = i % W
        row = czero + 2 * g + k // MYGRP
        col = (k % MYGRP) * 2 * L + 2 * lane
        labp[0, g, pl.ds(k * L, L)] = plsc.load_gather(lab2d, [row, col])
        labp[1, g, pl.ds(k * L, L)] = plsc.load_gather(lab2d, [row, col + 1])
        return 0
    lax.fori_loop(0, MYGRP * W, _mklabp, 0)

    zds = [pltpu.async_copy(zo_b, tab.at[lab2d.at[g]], semt)
           for g in range(NGRP)]
    for d in zds:
        d.wait()
    z1 = pltpu.async_copy(zo_hbm.at[1], zo_b, sem)
    z1.wait()
    plsc.subcore_barrier()

    ads = [pltpu.async_copy(zo_b, tab.at[lab2d.at[g]], semt, add=True)
           for g in range(NGRP)]
    for d in ads:
        d.wait()
    plsc.subcore_barrier()
    gds = [pltpu.async_copy(tab.at[lab2d.at[mygrp0 + g]],
                            S.at[2, pl.ds(g * G, G), :], semt)
           for g in range(MYGRP)]

    def _fire_feat(h):
        return [pltpu.async_copy(
            feat_hbm.at[pl.ds(tile_base // 2, SC_ROWS // 2),
                        pl.ds(p * N_FEAT + h * W, W)],
            f8.at[h % 2, p], sem) for p in range(2)]

    fd = _fire_feat(0)
    for d in gds:
        d.wait()

    ctwo = jnp.full((L,), 2, jnp.int32)

    def _coef(b, _):
        n16 = plsc.load_gather(S, [ctwo, b * L + lane, czero])
        d = 1.0 / (1.0 + n16)
        beta[pl.ds(b * L, L)] = 1.0 - ALPHA * n16 * d
        gamma[pl.ds(b * L, L)] = ALPHA * d
        return 0
    lax.fori_loop(0, MY_ROWS // L, _coef, 0)
    pltpu.sync_copy(beta, beta_hbm.at[pl.ds(my_base, MY_ROWS)])
    plsc.subcore_barrier()

    hsel = lane // W
    wsel = lane % W

    for h in range(NROUND):
        cur = h % 3
        if h % 2 == 0 and h + 1 < NROUND:
            fd_next = _fire_feat(h + 1)
        for d in fd:
            d.wait()
        ads = [pltpu.async_copy(f8.at[h % 2, p, pl.ds(g * G, G), :],
                                tab.at[labp.at[p, g]], semt, add=True)
               for p in range(2) for g in range(MYGRP)]
        for d in ads:
            d.wait()
        plsc.subcore_barrier()
        gds = [pltpu.async_copy(tab.at[lab2d.at[mygrp0 + g]],
                                S.at[cur, pl.ds(g * G, G), :], semt)
               for g in range(MYGRP)]
        for d in gds:
            d.wait()
        plsc.subcore_barrier()

        if h % 2 == 1:
            t = h // 2
            b1 = (2 * t) % 3
            b2 = (2 * t + 1) % 3
            bp = (2 * t - 1) % 3
            ssel = czero + b1 + hsel * (b2 - b1)
            psel = czero + bp + hsel * (b1 - bp)
            if t >= 2:
                pltpu.make_async_copy(
                    u_hbm.at[pl.ds(0, MY_ROWS), pl.ds(0, L)],
                    uacc.at[t % 2], semu).wait()

            def _comp(b, _):
                g16 = gamma[pl.ds(b * L, L)]
                for j in range(L):
                    s = b * L + j
                    srow = czero + s
                    f = plsc.load_gather(
                        f8, [hsel, czero + j % 2,
                             czero + cid * (MY_ROWS // 2) + b * W + j // 2,
                             wsel])
                    sc = plsc.load_gather(S, [ssel, srow, wsel])
                    sp = plsc.load_gather(S, [psel, srow, wsel])
                    uacc[t % 2, s, :] = f - g16[j] * (sc - sp)
                return 0
            lax.fori_loop(0, MY_ROWS // L, _comp, 0)
            pltpu.async_copy(
                uacc.at[t % 2],
                u_hbm.at[pl.ds(my_base, MY_ROWS), pl.ds(t * L, L)], semu)
            if h + 1 < NROUND:
                fd_next = _fire_feat(h + 1)
        if h + 1 < NROUND:
            fd = fd_next
    for t in range(2):
        pltpu.make_async_copy(
            u_hbm.at[pl.ds(0, MY_ROWS), pl.ds(0, L)],
            uacc.at[t], semu).wait()


def _body_b(u_hbm, beta_hbm, lab_hbm, cent_hbm, out_hbm,
            lab2d, c_loc, u_loc, racc, beta, sem):
    cid = lax.axis_index("c")
    sid = lax.axis_index("s")
    my_base = sid * SC_ROWS + cid * MY_ROWS

    lane = lax.iota(jnp.int32, L)
    czero = jnp.zeros((L,), jnp.int32)

    lds = [pltpu.async_copy(lab_hbm.at[pl.ds(my_base + g * G, G)],
                            lab2d.at[g], sem) for g in range(MYGRP)]
    ud = pltpu.async_copy(u_hbm.at[pl.ds(my_base, MY_ROWS)], u_loc, sem)
    bd = pltpu.async_copy(beta_hbm.at[pl.ds(my_base, MY_ROWS)], beta, sem)
    for d in lds:
        d.wait()
    cds = [pltpu.async_copy(cent_hbm.at[lab2d.at[g]],
                            c_loc.at[pl.ds(g * G, G)], sem)
           for g in range(MYGRP)]
    ud.wait()
    bd.wait()
    for d in cds:
        d.wait()

    def _comp(b, _):
        b16 = beta[pl.ds(b * L, L)]
        for j in range(L):
            s = b * L + j
            for t in range(NPAIR):
                u = u_loc[s, pl.ds(t * L, L)]
                c = c_loc[s, pl.ds(t * L, L)]
                d = u - b16[j] * c
                if t == 0:
                    racc[s, :] = d * d
                else:
                    racc[s, :] = racc[s, :] + d * d
        return 0
    lax.fori_loop(0, MY_ROWS // L, _comp, 0)

    def _fin(b, _):
        rows = b * L + lane
        acc = plsc.load_gather(racc, [rows, czero])
        for j in range(1, L):
            acc = acc + plsc.load_gather(racc,
                                         [rows, jnp.full((L,), j, jnp.int32)])
        beta[pl.ds(b * L, L)] = 0.5 * acc
        return 0
    lax.fori_loop(0, MY_ROWS // L, _fin, 0)
    pltpu.sync_copy(beta, out_hbm.at[pl.ds(my_base, MY_ROWS)])


_MESH = dict(core_axis_name="c", subcore_axis_name="s",
             num_cores=NC, num_subcores=NS)


@functools.cache
def _build_a():
    return functools.partial(
        pl.kernel,
        out_type=(jax.ShapeDtypeStruct((N_BATCH, N_FEAT), jnp.float32),
                  jax.ShapeDtypeStruct((N_BATCH,), jnp.float32)),
        compiler_params=pltpu.CompilerParams(use_tc_tiling_on_sc=False,
                                             needs_layout_passes=False),
        mesh=plsc.VectorSubcoreMesh(**_MESH),
        scratch_types=[
            pltpu.VMEM((NGRP, G), jnp.int32),
            pltpu.VMEM((2, MYGRP, G), jnp.int32),
            pltpu.VMEM((2, 2, SC_ROWS // 2, W), jnp.float32),
            pltpu.VMEM((3, MY_ROWS, W), jnp.float32),
            pltpu.VMEM((2, MY_ROWS, L), jnp.float32),
            pltpu.VMEM((MY_ROWS,), jnp.float32),
            pltpu.VMEM((MY_ROWS,), jnp.float32),
            pltpu.VMEM((G, W), jnp.float32),
            pltpu.VMEM_SHARED((N_CLASS, W), jnp.float32),
            pltpu.SemaphoreType.DMA,
            pltpu.SemaphoreType.DMA,
            pltpu.SemaphoreType.DMA,
        ],
    )(_body_a)


@functools.cache
def _build_b():
    return functools.partial(
        pl.kernel,
        out_type=jax.ShapeDtypeStruct((N_BATCH,), jnp.float32),
        compiler_params=pltpu.CompilerParams(use_tc_tiling_on_sc=False,
                                             needs_layout_passes=False),
        mesh=plsc.VectorSubcoreMesh(**_MESH),
        scratch_types=[
            pltpu.VMEM((MYGRP, G), jnp.int32),
            pltpu.VMEM((MY_ROWS, N_FEAT), jnp.float32),
            pltpu.VMEM((MY_ROWS, N_FEAT), jnp.float32),
            pltpu.VMEM((MY_ROWS, L), jnp.float32),
            pltpu.VMEM((MY_ROWS,), jnp.float32),
            pltpu.SemaphoreType.DMA,
        ],
    )(_body_b)


def kernel(features, labels, centers):
    labels = jnp.reshape(labels, (-1,)).astype(jnp.int32)
    zo = jnp.stack([jnp.zeros((G, W), jnp.float32),
                    jnp.ones((G, W), jnp.float32)])
    f2 = features.reshape(N_BATCH // 2, 2 * N_FEAT)
    u, bet = _build_a()(f2, labels, zo)
    out = _build_b()(u, bet, labels, centers)
    return jnp.reshape(out, (N_BATCH, 1))


# speedup vs baseline: 1.2250x; 1.0161x over previous
import functools

import jax
import jax.numpy as jnp
from jax import lax
from jax.experimental import pallas as pl
from jax.experimental.pallas import tpu as pltpu
from jax.experimental.pallas import tpu_sc as plsc

ALPHA = 0.5
N_CLASS = 100000
N_FEAT = 64
N_BATCH = 16384

L = 16
W = 8
NC = 2
NS = 16
G = 128
SC_ROWS = N_BATCH // NS
MY_ROWS = N_BATCH // (NC * NS)
NGRP = SC_ROWS // G
MYGRP = MY_ROWS // G
NROUND = N_FEAT // W
NPAIR = N_FEAT // L


def _body_a(feat_hbm, lab_hbm, zo_hbm, u_hbm, beta_hbm,
            lab2d, labp, f8, S, uacc, beta, gamma, zo_b, tab, sem, semt,
            semu):
    cid = lax.axis_index("c")
    sid = lax.axis_index("s")
    tile_base = sid * SC_ROWS
    my_base = tile_base + cid * MY_ROWS
    mygrp0 = cid * MYGRP

    lane = lax.iota(jnp.int32, L)
    czero = jnp.zeros((L,), jnp.int32)

    lds = [pltpu.async_copy(lab_hbm.at[pl.ds(tile_base + g * G, G)],
                            lab2d.at[g], sem) for g in range(NGRP)]
    z0 = pltpu.async_copy(zo_hbm.at[0], zo_b, sem)
    for d in lds:
        d.wait()
    z0.wait()

    def _mklabp(i, _):
        g = i // W
        k = i % W
        row = czero + 2 * g + k // MYGRP
        col = (k % MYGRP) * 2 * L + 2 * lane
        labp[0, g, pl.ds(k * L, L)] = plsc.load_gather(lab2d, [row, col])
        labp[1, g, pl.ds(k * L, L)] = plsc.load_gather(lab2d, [row, col + 1])
        return 0
    lax.fori_loop(0, MYGRP * W, _mklabp, 0)

    zds = [pltpu.async_copy(zo_b, tab.at[lab2d.at[g]], semt)
           for g in range(NGRP)]
    for d in zds:
        d.wait()
    z1 = pltpu.async_copy(zo_hbm.at[1], zo_b, sem)
    z1.wait()
    plsc.subcore_barrier()

    ads = [pltpu.async_copy(zo_b, tab.at[lab2d.at[g]], semt, add=True)
           for g in range(NGRP)]
    for d in ads:
        d.wait()
    plsc.subcore_barrier()
    gds = [pltpu.async_copy(tab.at[lab2d.at[mygrp0 + g]],
                            S.at[2, pl.ds(g * G, G), :], semt)
           for g in range(MYGRP)]

    def _fire_feat(h):
        return [pltpu.async_copy(
            feat_hbm.at[pl.ds(tile_base // 2, SC_ROWS // 2),
                        pl.ds(p * N_FEAT + h * W, W)],
            f8.at[h % 3, p], sem) for p in range(2)]

    fd = _fire_feat(0)
    for d in gds:
        d.wait()

    ctwo = jnp.full((L,), 2, jnp.int32)

    def _coef(b, _):
        n16 = plsc.load_gather(S, [ctwo, b * L + lane, czero])
        d = 1.0 / (1.0 + n16)
        beta[pl.ds(b * L, L)] = 1.0 - ALPHA * n16 * d
        gamma[pl.ds(b * L, L)] = ALPHA * d
        return 0
    lax.fori_loop(0, MY_ROWS // L, _coef, 0)
    pltpu.sync_copy(beta, beta_hbm.at[pl.ds(my_base, MY_ROWS)])
    plsc.subcore_barrier()

    hsel = lane // W
    wsel = lane % W

    for h in range(NROUND):
        cur = h % 3
        if h + 1 < NROUND:
            fd_next = _fire_feat(h + 1)
        for d in fd:
            d.wait()
        ads = [pltpu.async_copy(f8.at[h % 3, p, pl.ds(g * G, G), :],
                                tab.at[labp.at[p, g]], semt, add=True)
               for p in range(2) for g in range(MYGRP)]
        for d in ads:
            d.wait()
        plsc.subcore_barrier()
        gds = [pltpu.async_copy(tab.at[lab2d.at[mygrp0 + g]],
                                S.at[cur, pl.ds(g * G, G), :], semt)
               for g in range(MYGRP)]
        for d in gds:
            d.wait()
        plsc.subcore_barrier()

        if h % 2 == 1:
            t = h // 2
            b1 = (2 * t) % 3
            b2 = (2 * t + 1) % 3
            bp = (2 * t - 1) % 3
            ssel = czero + b1 + hsel * (b2 - b1)
            psel = czero + bp + hsel * (b1 - bp)
            fsel = czero + b1 + hsel * (b2 - b1)
            if t >= 2:
                pltpu.make_async_copy(
                    u_hbm.at[pl.ds(0, MY_ROWS), pl.ds(0, L)],
                    uacc.at[t % 2], semu).wait()

            def _comp(b, _):
                g16 = gamma[pl.ds(b * L, L)]
                for j in range(L):
                    s = b * L + j
                    srow = czero + s
                    f = plsc.load_gather(
                        f8, [fsel, czero + j % 2,
                             czero + cid * (MY_ROWS // 2) + b * W + j // 2,
                             wsel])
                    sc = plsc.load_gather(S, [ssel, srow, wsel])
                    sp = plsc.load_gather(S, [psel, srow, wsel])
                    uacc[t % 2, s, :] = f - g16[j] * (sc - sp)
                return 0
            lax.fori_loop(0, MY_ROWS // L, _comp, 0)
            pltpu.async_copy(
                uacc.at[t % 2],
                u_hbm.at[pl.ds(my_base, MY_ROWS), pl.ds(t * L, L)], semu)
        if h + 1 < NROUND:
            fd = fd_next
    for t in range(2):
        pltpu.make_async_copy(
            u_hbm.at[pl.ds(0, MY_ROWS), pl.ds(0, L)],
            uacc.at[t], semu).wait()


def _body_b(u_hbm, beta_hbm, lab_hbm, cent_hbm, out_hbm,
            lab2d, c_loc, u_loc, racc, beta, sem):
    cid = lax.axis_index("c")
    sid = lax.axis_index("s")
    my_base = sid * SC_ROWS + cid * MY_ROWS

    lane = lax.iota(jnp.int32, L)
    czero = jnp.zeros((L,), jnp.int32)

    lds = [pltpu.async_copy(lab_hbm.at[pl.ds(my_base + g * G, G)],
                            lab2d.at[g], sem) for g in range(MYGRP)]
    ud = pltpu.async_copy(u_hbm.at[pl.ds(my_base, MY_ROWS)], u_loc, sem)
    bd = pltpu.async_copy(beta_hbm.at[pl.ds(my_base, MY_ROWS)], beta, sem)
    for d in lds:
        d.wait()
    cds = [pltpu.async_copy(cent_hbm.at[lab2d.at[g]],
                            c_loc.at[pl.ds(g * G, G)], sem)
           for g in range(MYGRP)]
    ud.wait()
    bd.wait()
    for d in cds:
        d.wait()

    def _comp(b, _):
        b16 = beta[pl.ds(b * L, L)]
        for j in range(L):
            s = b * L + j
            for t in range(NPAIR):
                u = u_loc[s, pl.ds(t * L, L)]
                c = c_loc[s, pl.ds(t * L, L)]
                d = u - b16[j] * c
                if t == 0:
                    racc[s, :] = d * d
                else:
                    racc[s, :] = racc[s, :] + d * d
        return 0
    lax.fori_loop(0, MY_ROWS // L, _comp, 0)

    def _fin(b, _):
        rows = b * L + lane
        acc = plsc.load_gather(racc, [rows, czero])
        for j in range(1, L):
            acc = acc + plsc.load_gather(racc,
                                         [rows, jnp.full((L,), j, jnp.int32)])
        beta[pl.ds(b * L, L)] = 0.5 * acc
        return 0
    lax.fori_loop(0, MY_ROWS // L, _fin, 0)
    pltpu.sync_copy(beta, out_hbm.at[pl.ds(my_base, MY_ROWS)])


_MESH = dict(core_axis_name="c", subcore_axis_name="s",
             num_cores=NC, num_subcores=NS)


@functools.cache
def _build_a():
    return functools.partial(
        pl.kernel,
        out_type=(jax.ShapeDtypeStruct((N_BATCH, N_FEAT), jnp.float32),
                  jax.ShapeDtypeStruct((N_BATCH,), jnp.float32)),
        compiler_params=pltpu.CompilerParams(use_tc_tiling_on_sc=False,
                                             needs_layout_passes=False),
        mesh=plsc.VectorSubcoreMesh(**_MESH),
        scratch_types=[
            pltpu.VMEM((NGRP, G), jnp.int32),
            pltpu.VMEM((2, MYGRP, G), jnp.int32),
            pltpu.VMEM((3, 2, SC_ROWS // 2, W), jnp.float32),
            pltpu.VMEM((3, MY_ROWS, W), jnp.float32),
            pltpu.VMEM((2, MY_ROWS, L), jnp.float32),
            pltpu.VMEM((MY_ROWS,), jnp.float32),
            pltpu.VMEM((MY_ROWS,), jnp.float32),
            pltpu.VMEM((G, W), jnp.float32),
            pltpu.VMEM_SHARED((N_CLASS, W), jnp.float32),
            pltpu.SemaphoreType.DMA,
            pltpu.SemaphoreType.DMA,
            pltpu.SemaphoreType.DMA,
        ],
    )(_body_a)


@functools.cache
def _build_b():
    return functools.partial(
        pl.kernel,
        out_type=jax.ShapeDtypeStruct((N_BATCH,), jnp.float32),
        compiler_params=pltpu.CompilerParams(use_tc_tiling_on_sc=False,
                                             needs_layout_passes=False),
        mesh=plsc.VectorSubcoreMesh(**_MESH),
        scratch_types=[
            pltpu.VMEM((MYGRP, G), jnp.int32),
            pltpu.VMEM((MY_ROWS, N_FEAT), jnp.float32),
            pltpu.VMEM((MY_ROWS, N_FEAT), jnp.float32),
            pltpu.VMEM((MY_ROWS, L), jnp.float32),
            pltpu.VMEM((MY_ROWS,), jnp.float32),
            pltpu.SemaphoreType.DMA,
        ],
    )(_body_b)


def kernel(features, labels, centers):
    labels = jnp.reshape(labels, (-1,)).astype(jnp.int32)
    zo = jnp.stack([jnp.zeros((G, W), jnp.float32),
                    jnp.ones((G, W), jnp.float32)])
    f2 = features.reshape(N_BATCH // 2, 2 * N_FEAT)
    u, bet = _build_a()(f2, labels, zo)
    out = _build_b()(u, bet, labels, centers)
    return jnp.reshape(out, (N_BATCH, 1))
